# baseline (speedup 1.0000x reference)
import functools

import jax
import jax.numpy as jnp
from jax import lax
from jax.experimental import pallas as pl
from jax.experimental.pallas import tpu as pltpu
from jax.experimental.pallas import tpu_sc as plsc

N_NODES = 10000
N_PAD = 10240
D_FEAT = 128
N_EDGES = 320000

NC = 2
NS = 16
NW = NC * NS


def _seg_body(npad, d, epad, chunk, rchunk,
              src_hbm, dst_hbm, x_hbm, out_hbm,
              sidx_v, didx_v, rows_v, agg_sh,
              gsem0, gsem1, gsem2, gsem3, ssem,
              isem0, isem1, isem2, isem3, isem4, isem5):
    ept = epad // NW
    nchunk = ept // chunk
    assert nchunk % 12 == 5 and nchunk >= 29
    rpt = npad // NS
    nrc = rpt // rchunk
    assert rpt % rchunk == 0 and rchunk <= chunk

    c = lax.axis_index("c")
    s = lax.axis_index("s")
    wid = s * NC + c
    gsems = (gsem0, gsem1, gsem2, gsem3)
    isems = (isem0, isem1, isem2, isem3, isem4, isem5)
    ebase = wid * ept

    def _fire_idx(j, m):
        pltpu.async_copy(
            src_hbm.at[pl.ds(ebase + j * chunk, chunk)], sidx_v.at[m],
            isems[m])
        pltpu.async_copy(
            dst_hbm.at[pl.ds(ebase + j * chunk, chunk)], didx_v.at[m],
            isems[m])

    def _wait_idx(j, m):
        pltpu.make_async_copy(
            src_hbm.at[pl.ds(ebase + j * chunk, chunk)], sidx_v.at[m],
            isems[m]).wait()
        pltpu.make_async_copy(
            dst_hbm.at[pl.ds(ebase + j * chunk, chunk)], didx_v.at[m],
            isems[m]).wait()

    def _fire_g(mi, b):
        pltpu.async_copy(x_hbm.at[sidx_v.at[mi]], rows_v.at[b], gsems[b])

    def _wait_g(mi, b):
        pltpu.make_async_copy(
            x_hbm.at[sidx_v.at[mi]], rows_v.at[b], gsems[b]).wait()

    def _fire_s(mi, b):
        pltpu.async_copy(rows_v.at[b], agg_sh.at[didx_v.at[mi]], ssem,
                         add=True)

    def _drain_s(mi, b):
        pltpu.make_async_copy(
            rows_v.at[b], agg_sh.at[didx_v.at[mi]], ssem).wait()

    _fire_idx(0, 0)
    _fire_idx(1, 1)
    _fire_idx(2, 2)
    _fire_idx(3, 3)

    @pl.loop(0, rchunk)
    def _zbuf(i):
        for j in range(d // 16):
            rows_v[0, i, pl.ds(j * 16, 16)] = jnp.zeros((16,), jnp.float32)

    @pl.loop(0, nrc)
    def _zagg(i):
        pltpu.sync_copy(rows_v.at[0].at[pl.ds(0, rchunk)],
                        agg_sh.at[pl.ds(s * rpt + i * rchunk, rchunk)])

    plsc.subcore_barrier()

    def _step(j, m, drain=True, fidx=True, fg=True):
        if drain:
            _drain_s((m - 2) % 6, (m - 2) % 4)
        if fidx:
            _fire_idx(j + 4, (m + 4) % 6)
        if fg:
            _wait_idx(j + 2, (m + 2) % 6)
            _fire_g((m + 2) % 6, (m + 2) % 4)
        _wait_g(m % 6, m % 4)
        _fire_s(m % 6, m % 4)

    _wait_idx(0, 0)
    _fire_g(0, 0)
    _wait_idx(1, 1)
    _fire_g(1, 1)
    for j in range(13):
        _step(j, j, drain=(j >= 2))

    @pl.loop(13, nchunk - 4, step=12)
    def _edges(i):
        for b in range(12):
            _step(i + b, 13 + b)

    _step(nchunk - 4, nchunk - 4, fidx=False)
    _step(nchunk - 3, nchunk - 3, fidx=False)
    _step(nchunk - 2, nchunk - 2, fidx=False, fg=False)
    _step(nchunk - 1, nchunk - 1, fidx=False, fg=False)
    _drain_s((nchunk - 2) % 6, (nchunk - 2) % 4)
    _drain_s((nchunk - 1) % 6, (nchunk - 1) % 4)

    plsc.subcore_barrier()

    assert nrc % 2 == 0 and nrc >= 4

    def _wchunk(k, b, wait_prev):
        r0 = s * rpt + k * rchunk
        stage = rows_v.at[b].at[pl.ds(0, rchunk)]
        if wait_prev:
            rp = s * rpt + (k - 2) * rchunk
            pltpu.make_async_copy(
                rows_v.at[b].at[pl.ds(0, rchunk)],
                out_hbm.at[c, pl.ds(rp, rchunk)], gsems[b]).wait()
        pltpu.sync_copy(agg_sh.at[pl.ds(r0, rchunk)], stage)
        pltpu.async_copy(stage, out_hbm.at[c, pl.ds(r0, rchunk)], gsems[b])

    _wchunk(0, 0, False)
    _wchunk(1, 1, False)

    @pl.loop(2, nrc, step=2)
    def _wout(i):
        for b in range(2):
            _wchunk(i + b, b, True)

    pltpu.make_async_copy(
        rows_v.at[0].at[pl.ds(0, rchunk)],
        out_hbm.at[c, pl.ds(s * rpt + (nrc - 2) * rchunk, rchunk)],
        gsems[0]).wait()
    pltpu.make_async_copy(
        rows_v.at[1].at[pl.ds(0, rchunk)],
        out_hbm.at[c, pl.ds(s * rpt + (nrc - 1) * rchunk, rchunk)],
        gsems[1]).wait()


@functools.lru_cache(maxsize=None)
def _build_seg(npad, d, epad, chunk, rchunk, interpret=False):
    mesh = plsc.VectorSubcoreMesh(
        core_axis_name="c", subcore_axis_name="s",
        num_cores=NC, num_subcores=NS)
    return pl.kernel(
        functools.partial(_seg_body, npad, d, epad, chunk, rchunk),
        out_type=jax.ShapeDtypeStruct((NC, npad, d), jnp.float32),
        mesh=mesh,
        scratch_types=[
            pltpu.VMEM((6, chunk), jnp.int32),
            pltpu.VMEM((6, chunk), jnp.int32),
            pltpu.VMEM((4, chunk, d), jnp.float32),
            pltpu.VMEM_SHARED((npad, d), jnp.float32),
        ] + [pltpu.SemaphoreType.DMA] * 11,
        interpret=interpret,
    )


def _tc_root_body(h_ref, wt_ref, b_ref, o_ref):
    y = jnp.dot(h_ref[...], wt_ref[...], preferred_element_type=jnp.float32)
    o_ref[...] = y + b_ref[...]


def _tc_comb_body(agg_ref, rt_ref, wr_ref, o_ref):
    a = agg_ref[0] + agg_ref[1]
    y = jnp.dot(a, wr_ref[...], preferred_element_type=jnp.float32)
    o_ref[...] = jnp.maximum(y + rt_ref[...], 0.0)


@functools.lru_cache(maxsize=None)
def _build_tc_root(n, d, blk, interpret=False):
    return pl.pallas_call(
        _tc_root_body,
        grid=(n // blk,),
        in_specs=[
            pl.BlockSpec((blk, d), lambda i: (i, 0)),
            pl.BlockSpec((d, d), lambda i: (0, 0)),
            pl.BlockSpec((1, d), lambda i: (0, 0)),
        ],
        out_specs=pl.BlockSpec((blk, d), lambda i: (i, 0)),
        out_shape=jax.ShapeDtypeStruct((n, d), jnp.float32),
        interpret=interpret,
    )


@functools.lru_cache(maxsize=None)
def _build_tc_comb(n, d, blk, interpret=False):
    return pl.pallas_call(
        _tc_comb_body,
        grid=(n // blk,),
        in_specs=[
            pl.BlockSpec((NC, blk, d), lambda i: (0, i, 0)),
            pl.BlockSpec((blk, d), lambda i: (i, 0)),
            pl.BlockSpec((d, d), lambda i: (0, 0)),
        ],
        out_specs=pl.BlockSpec((blk, d), lambda i: (i, 0)),
        out_shape=jax.ShapeDtypeStruct((n, d), jnp.float32),
        interpret=interpret,
    )


def kernel(x, edge_index, W1_rel, b1_rel, W1_root, W2_rel, b2_rel, W2_root,
           W3_rel, b3_rel, W3_root):
    chunk = 80
    nchunk = N_EDGES // NW // chunk
    ei = edge_index.astype(jnp.int32)
    src, dst = ei[0], ei[1]
    seg = _build_seg(N_PAD, D_FEAT, N_EDGES, chunk, 80)
    tcr = _build_tc_root(N_NODES, D_FEAT, 1000)
    tcc = _build_tc_comb(N_NODES, D_FEAT, 1000)
    h = x
    for wr, b, wt in ((W1_rel, b1_rel, W1_root),
                      (W2_rel, b2_rel, W2_root),
                      (W3_rel, b3_rel, W3_root)):
        rt = tcr(h, wt, b.reshape(1, D_FEAT))
        agg2 = seg(src, dst, h)
        h = tcc(agg2, rt, wr)
    return h.reshape(1, N_NODES, D_FEAT)


# speedup vs baseline: 1.0014x; 1.0014x over previous
import functools

import jax
import jax.numpy as jnp
from jax import lax
from jax.experimental import pallas as pl
from jax.experimental.pallas import tpu as pltpu
from jax.experimental.pallas import tpu_sc as plsc

N_NODES = 10000
N_PAD = 10240
D_FEAT = 128
N_EDGES = 320000

NC = 2
NS = 16
NW = NC * NS


def _seg_body(npad, d, epad, chunk, rchunk,
              src_hbm, dst_hbm, x_hbm, out_hbm,
              sidx_v, didx_v, rows_v, agg_sh,
              gsem0, gsem1, gsem2, gsem3, ssem,
              isem0, isem1, isem2, isem3, isem4, isem5):
    ept = epad // NW
    nchunk = ept // chunk
    assert nchunk % 12 == 5 and nchunk >= 29
    rpt = npad // NS
    nrc = rpt // rchunk
    assert rpt % rchunk == 0 and rchunk <= chunk

    c = lax.axis_index("c")
    s = lax.axis_index("s")
    wid = s * NC + c
    gsems = (gsem0, gsem1, gsem2, gsem3)
    isems = (isem0, isem1, isem2, isem3, isem4, isem5)
    ebase = wid * ept

    def _fire_idx(j, m):
        pltpu.async_copy(
            src_hbm.at[pl.ds(ebase + j * chunk, chunk)], sidx_v.at[m],
            isems[m])
        pltpu.async_copy(
            dst_hbm.at[pl.ds(ebase + j * chunk, chunk)], didx_v.at[m],
            isems[m])

    def _wait_idx(j, m):
        pltpu.make_async_copy(
            src_hbm.at[pl.ds(ebase + j * chunk, chunk)], sidx_v.at[m],
            isems[m]).wait()
        pltpu.make_async_copy(
            dst_hbm.at[pl.ds(ebase + j * chunk, chunk)], didx_v.at[m],
            isems[m]).wait()

    def _fire_g(mi, b):
        pltpu.async_copy(x_hbm.at[sidx_v.at[mi]], rows_v.at[b], gsems[b])

    def _wait_g(mi, b):
        pltpu.make_async_copy(
            x_hbm.at[sidx_v.at[mi]], rows_v.at[b], gsems[b]).wait()

    def _fire_s(mi, b):
        pltpu.async_copy(rows_v.at[b], agg_sh.at[didx_v.at[mi]], ssem,
                         add=True)

    def _drain_s(mi, b):
        pltpu.make_async_copy(
            rows_v.at[b], agg_sh.at[didx_v.at[mi]], ssem).wait()

    _fire_idx(0, 0)
    _fire_idx(1, 1)
    _fire_idx(2, 2)
    _fire_idx(3, 3)

    @pl.loop(0, rchunk)
    def _zbuf(i):
        for j in range(d // 16):
            rows_v[0, i, pl.ds(j * 16, 16)] = jnp.zeros((16,), jnp.float32)

    @pl.loop(0, nrc)
    def _zagg(i):
        pltpu.sync_copy(rows_v.at[0].at[pl.ds(0, rchunk)],
                        agg_sh.at[pl.ds(s * rpt + i * rchunk, rchunk)])

    plsc.subcore_barrier()

    def _step(j, m, drain=True, fidx=True, fg=True):
        if drain:
            _drain_s((m - 2) % 6, (m - 2) % 4)
        if fidx:
            _fire_idx(j + 4, (m + 4) % 6)
        if fg:
            _wait_idx(j + 2, (m + 2) % 6)
            _fire_g((m + 2) % 6, (m + 2) % 4)
        _wait_g(m % 6, m % 4)
        _fire_s(m % 6, m % 4)

    _wait_idx(0, 0)
    _fire_g(0, 0)
    _wait_idx(1, 1)
    _fire_g(1, 1)
    for j in range(13):
        _step(j, j, drain=(j >= 2))

    @pl.loop(13, nchunk - 4, step=12)
    def _edges(i):
        for b in range(12):
            _step(i + b, 13 + b)

    _step(nchunk - 4, nchunk - 4, fidx=False)
    _step(nchunk - 3, nchunk - 3, fidx=False)
    _step(nchunk - 2, nchunk - 2, fidx=False, fg=False)
    _step(nchunk - 1, nchunk - 1, fidx=False, fg=False)
    _drain_s((nchunk - 2) % 6, (nchunk - 2) % 4)
    _drain_s((nchunk - 1) % 6, (nchunk - 1) % 4)

    plsc.subcore_barrier()

    assert nrc % 2 == 0 and nrc >= 4

    def _wchunk(k, b, wait_prev):
        r0 = s * rpt + k * rchunk
        stage = rows_v.at[b].at[pl.ds(0, rchunk)]
        if wait_prev:
            rp = s * rpt + (k - 2) * rchunk
            pltpu.make_async_copy(
                rows_v.at[b].at[pl.ds(0, rchunk)],
                out_hbm.at[c, pl.ds(rp, rchunk)], gsems[b]).wait()
        pltpu.sync_copy(agg_sh.at[pl.ds(r0, rchunk)], stage)
        pltpu.async_copy(stage, out_hbm.at[c, pl.ds(r0, rchunk)], gsems[b])

    _wchunk(0, 0, False)
    _wchunk(1, 1, False)

    @pl.loop(2, nrc, step=2)
    def _wout(i):
        for b in range(2):
            _wchunk(i + b, b, True)

    pltpu.make_async_copy(
        rows_v.at[0].at[pl.ds(0, rchunk)],
        out_hbm.at[c, pl.ds(s * rpt + (nrc - 2) * rchunk, rchunk)],
        gsems[0]).wait()
    pltpu.make_async_copy(
        rows_v.at[1].at[pl.ds(0, rchunk)],
        out_hbm.at[c, pl.ds(s * rpt + (nrc - 1) * rchunk, rchunk)],
        gsems[1]).wait()


@functools.lru_cache(maxsize=None)
def _build_seg(npad, d, epad, chunk, rchunk):
    mesh = plsc.VectorSubcoreMesh(
        core_axis_name="c", subcore_axis_name="s",
        num_cores=NC, num_subcores=NS)
    return pl.kernel(
        functools.partial(_seg_body, npad, d, epad, chunk, rchunk),
        out_type=jax.ShapeDtypeStruct((NC, npad, d), jnp.float32),
        mesh=mesh,
        scratch_types=[
            pltpu.VMEM((6, chunk), jnp.int32),
            pltpu.VMEM((6, chunk), jnp.int32),
            pltpu.VMEM((4, chunk, d), jnp.float32),
            pltpu.VMEM_SHARED((npad, d), jnp.float32),
        ] + [pltpu.SemaphoreType.DMA] * 11,
    )


def _tc_body(agg_ref, h_ref, wr_ref, b_ref, wt_ref, o_ref):
    a = agg_ref[0] + agg_ref[1]
    y = jnp.dot(a, wr_ref[...], preferred_element_type=jnp.float32)
    y = y + jnp.dot(h_ref[...], wt_ref[...], preferred_element_type=jnp.float32)
    o_ref[...] = jnp.maximum(y + b_ref[...], 0.0)


@functools.lru_cache(maxsize=None)
def _build_tc(n, d, blk):
    return pl.pallas_call(
        _tc_body,
        grid=(n // blk,),
        in_specs=[
            pl.BlockSpec((NC, blk, d), lambda i: (0, i, 0)),
            pl.BlockSpec((blk, d), lambda i: (i, 0)),
            pl.BlockSpec((d, d), lambda i: (0, 0)),
            pl.BlockSpec((1, d), lambda i: (0, 0)),
            pl.BlockSpec((d, d), lambda i: (0, 0)),
        ],
        out_specs=pl.BlockSpec((blk, d), lambda i: (i, 0)),
        out_shape=jax.ShapeDtypeStruct((n, d), jnp.float32),
    )


def kernel(x, edge_index, W1_rel, b1_rel, W1_root, W2_rel, b2_rel, W2_root,
           W3_rel, b3_rel, W3_root):
    chunk = 80
    ei = edge_index.astype(jnp.int32)
    src, dst = ei[0], ei[1]
    seg = _build_seg(N_PAD, D_FEAT, N_EDGES, chunk, 80)
    tc = _build_tc(N_NODES, D_FEAT, 1000)
    h = x
    for wr, b, wt in ((W1_rel, b1_rel, W1_root),
                      (W2_rel, b2_rel, W2_root),
                      (W3_rel, b3_rel, W3_root)):
        agg2 = seg(src, dst, h)
        h = tc(agg2, h, wr, b.reshape(1, D_FEAT), wt)
    return h.reshape(1, N_NODES, D_FEAT)
